# baseline (speedup 1.0000x reference)
import jax
import jax.numpy as jnp
from jax import lax
from jax.experimental import pallas as pl
from jax.experimental.pallas import tpu as pltpu

_NC, _NF, _DX, _DS, _B = 4096, 16384, 64, 64, 16
_DIN, _DOUT, _K = 128, 128, 3
_T = 128
_NT = _NF // _T
_W = 256
_MAXCH = _NC // _W
_INF = float("inf")


def _body(pos_t_ref, batch_row_ref, x_hi_ref, x_lo_ref, pos_skip_ref,
          batch_col_ref,
          x_skip_ref, w1a_ref, w1b_ref, p1_ref, w2_ref, p2_ref,
          cbase_ref, nch_ref, out_ref, h1_ref, d2c_ref):

    def tile_knn_mlp1(t, carry):
        s1, q1 = carry
        o = pl.multiple_of(t * _T, _T)
        ps = pos_skip_ref[pl.ds(o, _T), :]
        bs = batch_col_ref[pl.ds(o, _T), :]
        psn = jnp.sum(ps * ps, axis=1, keepdims=True)
        ps_bf = ps.astype(jnp.bfloat16)
        base0 = cbase_ref[t]
        nch = nch_ref[t]

        def d2_chunk(c):
            cb = pl.multiple_of(base0 + c * _W, _W)
            pt = pos_t_ref[:, pl.ds(cb, _W)]
            br = batch_row_ref[:, pl.ds(cb, _W)]
            cross = jnp.dot(ps_bf, pt.astype(jnp.bfloat16),
                            preferred_element_type=jnp.float32)
            pn = jnp.sum(pt * pt, axis=0, keepdims=True)
            d2 = jnp.maximum((psn + pn) - 2.0 * cross, 0.0)
            return jnp.where(bs != br, _INF, d2), cb

        def p1(c, vs):
            d2, _ = d2_chunk(c)
            d2c_ref[:, pl.ds(c * _W, _W)] = d2
            m1 = jnp.min(d2, axis=1, keepdims=True)
            d2 = jnp.where(d2 <= m1, _INF, d2)
            m2 = jnp.min(d2, axis=1, keepdims=True)
            d2 = jnp.where(d2 <= m2, _INF, d2)
            m3 = jnp.min(d2, axis=1, keepdims=True)
            cur = list(vs) + [m1, m2, m3]
            out = []
            for _ in range(_K):
                m = cur[0]
                for v in cur[1:]:
                    m = jnp.minimum(m, v)
                out.append(m)
                cur = [jnp.where(v <= m, _INF, v) for v in cur]
            return tuple(out)

        ful = jnp.full((_T, 1), _INF, jnp.float32)
        _, _, v3 = lax.fori_loop(0, nch, p1, (ful, ful, ful))

        def p2(c, acc):
            ynum, den = acc
            cb = pl.multiple_of(base0 + c * _W, _W)
            d2 = d2c_ref[:, pl.ds(c * _W, _W)]
            w = jnp.where(d2 <= v3, 1.0 / jnp.maximum(d2, 1e-16), 0.0)
            den = den + jnp.sum(w, axis=1, keepdims=True)
            w_hi = w.astype(jnp.bfloat16)
            w_lo = (w - w_hi.astype(jnp.float32)).astype(jnp.bfloat16)
            xh = x_hi_ref[pl.ds(cb, _W), :]
            xl = x_lo_ref[pl.ds(cb, _W), :]
            ynum = (ynum
                    + jnp.dot(w_hi, xh, preferred_element_type=jnp.float32)
                    + jnp.dot(w_hi, xl, preferred_element_type=jnp.float32)
                    + jnp.dot(w_lo, xh, preferred_element_type=jnp.float32))
            return ynum, den

        ynum, den = lax.fori_loop(
            0, nch, p2, (jnp.zeros((_T, _DX), jnp.float32),
                         jnp.zeros((_T, 1), jnp.float32)))
        y = ynum / den

        xs = x_skip_ref[pl.ds(o, _T), :]
        h1 = (jnp.dot(y, w1a_ref[...], preferred_element_type=jnp.float32)
              + jnp.dot(xs, w1b_ref[...], preferred_element_type=jnp.float32)
              + p1_ref[0:1, :])
        h1_ref[pl.ds(o, _T), :] = h1
        return (s1 + jnp.sum(h1, axis=0, keepdims=True),
                q1 + jnp.sum(h1 * h1, axis=0, keepdims=True))

    z = jnp.zeros((1, _DOUT), jnp.float32)
    s1, q1 = lax.fori_loop(0, _NT, tile_knn_mlp1, (z, z))
    mu1 = s1 / _NF
    var1 = q1 / _NF - mu1 * mu1
    sc1 = p1_ref[1:2, :] * lax.rsqrt(var1 + 1e-5)
    sh1 = p1_ref[2:3, :] - mu1 * sc1

    def tile_mlp2(t, carry):
        s2, q2 = carry
        o = pl.multiple_of(t * _T, _T)
        h1 = h1_ref[pl.ds(o, _T), :]
        zrel = jnp.maximum(h1 * sc1 + sh1, 0.0)
        h2 = (jnp.dot(zrel, w2_ref[...], preferred_element_type=jnp.float32)
              + p2_ref[0:1, :])
        out_ref[pl.ds(o, _T), :] = h2
        return (s2 + jnp.sum(h2, axis=0, keepdims=True),
                q2 + jnp.sum(h2 * h2, axis=0, keepdims=True))

    s2, q2 = lax.fori_loop(0, _NT, tile_mlp2, (z, z))
    mu2 = s2 / _NF
    var2 = q2 / _NF - mu2 * mu2
    sc2 = p2_ref[1:2, :] * lax.rsqrt(var2 + 1e-5)
    sh2 = p2_ref[2:3, :] - mu2 * sc2

    def tile_bn2(t, _):
        o = pl.multiple_of(t * _T, _T)
        h2 = out_ref[pl.ds(o, _T), :]
        out_ref[pl.ds(o, _T), :] = jnp.maximum(h2 * sc2 + sh2, 0.0)
        return 0

    lax.fori_loop(0, _NT, tile_bn2, 0)


def kernel(x, pos, batch, x_skip, pos_skip, batch_skip,
           W1, b1, g1, be1, W2, b2, g2, be2):
    pos_t = pos.T
    batch_i = batch.astype(jnp.int32)
    batch_row = batch_i.reshape(1, _NC)
    batch_col = batch_skip.astype(jnp.int32).reshape(_NF, 1)
    x_hi = x.astype(jnp.bfloat16)
    x_lo = (x - x_hi.astype(jnp.float32)).astype(jnp.bfloat16)
    w1a, w1b = W1[:_DX], W1[_DX:]
    p1 = jnp.stack([b1, g1, be1])
    p2 = jnp.stack([b2, g2, be2])

    tids = jnp.arange(_NT)
    blo = batch_skip[tids * _T]
    bhi = batch_skip[tids * _T + (_T - 1)]
    clo = jnp.searchsorted(batch_i, blo.astype(jnp.int32), side="left")
    chi = jnp.searchsorted(batch_i, bhi.astype(jnp.int32), side="right")
    cbase = ((clo // _W) * _W).astype(jnp.int32)
    nch = ((chi.astype(jnp.int32) - cbase + _W - 1) // _W)

    h = pl.pallas_call(
        _body,
        out_shape=jax.ShapeDtypeStruct((_NF, _DOUT), jnp.float32),
        in_specs=[pl.BlockSpec(memory_space=pltpu.VMEM)] * 12
        + [pl.BlockSpec(memory_space=pltpu.SMEM)] * 2,
        out_specs=pl.BlockSpec(memory_space=pltpu.VMEM),
        scratch_shapes=[pltpu.VMEM((_NF, _DOUT), jnp.float32),
                        pltpu.VMEM((_T, _MAXCH * _W), jnp.float32)],
    )(pos_t, batch_row, x_hi, x_lo, pos_skip, batch_col, x_skip,
      w1a, w1b, p1, W2, p2, cbase, nch)
    return (h, pos_skip, batch_skip)


# speedup vs baseline: 1.7127x; 1.7127x over previous
import jax
import jax.numpy as jnp
from jax import lax
from jax.experimental import pallas as pl
from jax.experimental.pallas import tpu as pltpu

_NC, _NF, _DX, _DS, _B = 4096, 16384, 64, 64, 16
_DIN, _DOUT, _K = 128, 128, 3
_T = 512
_NT = _NF // _T
_W = 256
_MAXCH = _NC // _W
_INF = float("inf")


def _body(pos_t_ref, batch_row_ref, x_hi_ref, x_lo_ref, pos_skip_ref,
          batch_col_ref,
          x_skip_ref, w1a_ref, w1b_ref, p1_ref, w2_ref, p2_ref,
          cbase_ref, nch_ref, out_ref, h1_ref, d2c_ref):

    def tile_knn_mlp1(t, carry):
        s1, q1 = carry
        o = pl.multiple_of(t * _T, _T)
        ps = pos_skip_ref[pl.ds(o, _T), :]
        bs = batch_col_ref[pl.ds(o, _T), :]
        psn = jnp.sum(ps * ps, axis=1, keepdims=True)
        ps_bf = ps.astype(jnp.bfloat16)
        base0 = cbase_ref[t]
        nch = nch_ref[t]

        def d2_chunk(c):
            cb = pl.multiple_of(base0 + c * _W, _W)
            pt = pos_t_ref[:, pl.ds(cb, _W)]
            br = batch_row_ref[:, pl.ds(cb, _W)]
            cross = jnp.dot(ps_bf, pt.astype(jnp.bfloat16),
                            preferred_element_type=jnp.float32)
            pn = jnp.sum(pt * pt, axis=0, keepdims=True)
            d2 = jnp.maximum((psn + pn) - 2.0 * cross, 0.0)
            return jnp.where(bs != br, _INF, d2), cb

        def p1(c, vs):
            d2, _ = d2_chunk(c)
            d2c_ref[:, pl.ds(c * _W, _W)] = d2
            m1 = jnp.min(d2, axis=1, keepdims=True)
            d2 = jnp.where(d2 <= m1, _INF, d2)
            m2 = jnp.min(d2, axis=1, keepdims=True)
            d2 = jnp.where(d2 <= m2, _INF, d2)
            m3 = jnp.min(d2, axis=1, keepdims=True)
            cur = list(vs) + [m1, m2, m3]
            out = []
            for _ in range(_K):
                m = cur[0]
                for v in cur[1:]:
                    m = jnp.minimum(m, v)
                out.append(m)
                cur = [jnp.where(v <= m, _INF, v) for v in cur]
            return tuple(out)

        ful = jnp.full((_T, 1), _INF, jnp.float32)
        _, _, v3 = lax.fori_loop(0, nch, p1, (ful, ful, ful))

        def p2(c, acc):
            ynum, den = acc
            cb = pl.multiple_of(base0 + c * _W, _W)
            d2 = d2c_ref[:, pl.ds(c * _W, _W)]
            w = jnp.where(d2 <= v3, 1.0 / jnp.maximum(d2, 1e-16), 0.0)
            den = den + jnp.sum(w, axis=1, keepdims=True)
            w_hi = w.astype(jnp.bfloat16)
            w_lo = (w - w_hi.astype(jnp.float32)).astype(jnp.bfloat16)
            xh = x_hi_ref[pl.ds(cb, _W), :]
            xl = x_lo_ref[pl.ds(cb, _W), :]
            ynum = (ynum
                    + jnp.dot(w_hi, xh, preferred_element_type=jnp.float32)
                    + jnp.dot(w_hi, xl, preferred_element_type=jnp.float32)
                    + jnp.dot(w_lo, xh, preferred_element_type=jnp.float32))
            return ynum, den

        ynum, den = lax.fori_loop(
            0, nch, p2, (jnp.zeros((_T, _DX), jnp.float32),
                         jnp.zeros((_T, 1), jnp.float32)))
        y = ynum / den

        xs = x_skip_ref[pl.ds(o, _T), :]
        h1 = (jnp.dot(y, w1a_ref[...], preferred_element_type=jnp.float32)
              + jnp.dot(xs, w1b_ref[...], preferred_element_type=jnp.float32)
              + p1_ref[0:1, :])
        h1_ref[pl.ds(o, _T), :] = h1
        return (s1 + jnp.sum(h1, axis=0, keepdims=True),
                q1 + jnp.sum(h1 * h1, axis=0, keepdims=True))

    z = jnp.zeros((1, _DOUT), jnp.float32)
    s1, q1 = lax.fori_loop(0, _NT, tile_knn_mlp1, (z, z))
    mu1 = s1 / _NF
    var1 = q1 / _NF - mu1 * mu1
    sc1 = p1_ref[1:2, :] * lax.rsqrt(var1 + 1e-5)
    sh1 = p1_ref[2:3, :] - mu1 * sc1

    def tile_mlp2(t, carry):
        s2, q2 = carry
        o = pl.multiple_of(t * _T, _T)
        h1 = h1_ref[pl.ds(o, _T), :]
        zrel = jnp.maximum(h1 * sc1 + sh1, 0.0)
        h2 = (jnp.dot(zrel, w2_ref[...], preferred_element_type=jnp.float32)
              + p2_ref[0:1, :])
        out_ref[pl.ds(o, _T), :] = h2
        return (s2 + jnp.sum(h2, axis=0, keepdims=True),
                q2 + jnp.sum(h2 * h2, axis=0, keepdims=True))

    s2, q2 = lax.fori_loop(0, _NT, tile_mlp2, (z, z))
    mu2 = s2 / _NF
    var2 = q2 / _NF - mu2 * mu2
    sc2 = p2_ref[1:2, :] * lax.rsqrt(var2 + 1e-5)
    sh2 = p2_ref[2:3, :] - mu2 * sc2

    def tile_bn2(t, _):
        o = pl.multiple_of(t * _T, _T)
        h2 = out_ref[pl.ds(o, _T), :]
        out_ref[pl.ds(o, _T), :] = jnp.maximum(h2 * sc2 + sh2, 0.0)
        return 0

    lax.fori_loop(0, _NT, tile_bn2, 0)


def kernel(x, pos, batch, x_skip, pos_skip, batch_skip,
           W1, b1, g1, be1, W2, b2, g2, be2):
    pos_t = pos.T
    batch_i = batch.astype(jnp.int32)
    batch_row = batch_i.reshape(1, _NC)
    batch_col = batch_skip.astype(jnp.int32).reshape(_NF, 1)
    x_hi = x.astype(jnp.bfloat16)
    x_lo = (x - x_hi.astype(jnp.float32)).astype(jnp.bfloat16)
    w1a, w1b = W1[:_DX], W1[_DX:]
    p1 = jnp.stack([b1, g1, be1])
    p2 = jnp.stack([b2, g2, be2])

    tids = jnp.arange(_NT)
    blo = batch_skip[tids * _T]
    bhi = batch_skip[tids * _T + (_T - 1)]
    clo = jnp.searchsorted(batch_i, blo.astype(jnp.int32), side="left")
    chi = jnp.searchsorted(batch_i, bhi.astype(jnp.int32), side="right")
    cbase = ((clo // _W) * _W).astype(jnp.int32)
    nch = ((chi.astype(jnp.int32) - cbase + _W - 1) // _W)

    h = pl.pallas_call(
        _body,
        out_shape=jax.ShapeDtypeStruct((_NF, _DOUT), jnp.float32),
        in_specs=[pl.BlockSpec(memory_space=pltpu.VMEM)] * 12
        + [pl.BlockSpec(memory_space=pltpu.SMEM)] * 2,
        out_specs=pl.BlockSpec(memory_space=pltpu.VMEM),
        scratch_shapes=[pltpu.VMEM((_NF, _DOUT), jnp.float32),
                        pltpu.VMEM((_T, _MAXCH * _W), jnp.float32)],
    )(pos_t, batch_row, x_hi, x_lo, pos_skip, batch_col, x_skip,
      w1a, w1b, p1, W2, p2, cbase, nch)
    return (h, pos_skip, batch_skip)


# speedup vs baseline: 1.7231x; 1.0061x over previous
import jax
import jax.numpy as jnp
from jax import lax
from jax.experimental import pallas as pl
from jax.experimental.pallas import tpu as pltpu

_NC, _NF, _DX, _DS, _B = 4096, 16384, 64, 64, 16
_DIN, _DOUT, _K = 128, 128, 3
_T = 512
_NT = _NF // _T
_W = 256
_MAXCH = _NC // _W
_INF = float("inf")


def _body(pos_t_ref, batch_row_ref, x_hi_ref, x_lo_ref, pos_skip_ref,
          batch_col_ref,
          x_skip_ref, w1a_ref, w1b_ref, p1_ref, w2_ref, p2_ref,
          cbase_ref, nch_ref, out_ref, h1_ref, d2c_ref):

    def tile_knn_mlp1(t, carry):
        s1, q1 = carry
        o = pl.multiple_of(t * _T, _T)
        ps = pos_skip_ref[pl.ds(o, _T), :]
        bs = batch_col_ref[pl.ds(o, _T), :]
        psn = jnp.sum(ps * ps, axis=1, keepdims=True)
        ps_bf = ps.astype(jnp.bfloat16)
        base0 = cbase_ref[t]
        nch = nch_ref[t]

        def d2_chunk(c):
            cb = pl.multiple_of(base0 + c * _W, _W)
            pt = pos_t_ref[:, pl.ds(cb, _W)]
            br = batch_row_ref[:, pl.ds(cb, _W)]
            cross = jnp.dot(ps_bf, pt.astype(jnp.bfloat16),
                            preferred_element_type=jnp.float32)
            pn = jnp.sum(pt * pt, axis=0, keepdims=True)
            d2 = jnp.maximum((psn + pn) - 2.0 * cross, 0.0)
            return jnp.where(bs != br, _INF, d2), cb

        def p1(c, vs):
            d2, _ = d2_chunk(c)
            d2c_ref[:, pl.ds(c * _W, _W)] = d2
            m1 = jnp.min(d2, axis=1, keepdims=True)
            d2 = jnp.where(d2 <= m1, _INF, d2)
            m2 = jnp.min(d2, axis=1, keepdims=True)
            d2 = jnp.where(d2 <= m2, _INF, d2)
            m3 = jnp.min(d2, axis=1, keepdims=True)
            cur = list(vs) + [m1, m2, m3]
            out = []
            for _ in range(_K):
                m = cur[0]
                for v in cur[1:]:
                    m = jnp.minimum(m, v)
                out.append(m)
                cur = [jnp.where(v <= m, _INF, v) for v in cur]
            return tuple(out)

        ful = jnp.full((_T, 1), _INF, jnp.float32)
        v1, v2, v3 = lax.fori_loop(0, nch, p1, (ful, ful, ful))
        den = (1.0 / jnp.maximum(v1, 1e-16)
               + 1.0 / jnp.maximum(v2, 1e-16)
               + 1.0 / jnp.maximum(v3, 1e-16))

        def p2(c, ynum):
            cb = pl.multiple_of(base0 + c * _W, _W)
            d2 = d2c_ref[:, pl.ds(c * _W, _W)]
            w = jnp.where(d2 <= v3, 1.0 / jnp.maximum(d2, 1e-16), 0.0)
            w_hi = w.astype(jnp.bfloat16)
            w_lo = (w - w_hi.astype(jnp.float32)).astype(jnp.bfloat16)
            xh = x_hi_ref[pl.ds(cb, _W), :]
            xl = x_lo_ref[pl.ds(cb, _W), :]
            return (ynum
                    + jnp.dot(w_hi, xh, preferred_element_type=jnp.float32)
                    + jnp.dot(w_hi, xl, preferred_element_type=jnp.float32)
                    + jnp.dot(w_lo, xh, preferred_element_type=jnp.float32))

        ynum = lax.fori_loop(0, nch, p2, jnp.zeros((_T, _DX), jnp.float32))
        y = ynum / den

        xs = x_skip_ref[pl.ds(o, _T), :]
        h1 = (jnp.dot(y, w1a_ref[...], preferred_element_type=jnp.float32)
              + jnp.dot(xs, w1b_ref[...], preferred_element_type=jnp.float32)
              + p1_ref[0:1, :])
        h1_ref[pl.ds(o, _T), :] = h1
        return (s1 + jnp.sum(h1, axis=0, keepdims=True),
                q1 + jnp.sum(h1 * h1, axis=0, keepdims=True))

    z = jnp.zeros((1, _DOUT), jnp.float32)
    s1, q1 = lax.fori_loop(0, _NT, tile_knn_mlp1, (z, z))
    mu1 = s1 / _NF
    var1 = q1 / _NF - mu1 * mu1
    sc1 = p1_ref[1:2, :] * lax.rsqrt(var1 + 1e-5)
    sh1 = p1_ref[2:3, :] - mu1 * sc1

    def tile_mlp2(t, carry):
        s2, q2 = carry
        o = pl.multiple_of(t * _T, _T)
        h1 = h1_ref[pl.ds(o, _T), :]
        zrel = jnp.maximum(h1 * sc1 + sh1, 0.0)
        h2 = (jnp.dot(zrel, w2_ref[...], preferred_element_type=jnp.float32)
              + p2_ref[0:1, :])
        out_ref[pl.ds(o, _T), :] = h2
        return (s2 + jnp.sum(h2, axis=0, keepdims=True),
                q2 + jnp.sum(h2 * h2, axis=0, keepdims=True))

    s2, q2 = lax.fori_loop(0, _NT, tile_mlp2, (z, z))
    mu2 = s2 / _NF
    var2 = q2 / _NF - mu2 * mu2
    sc2 = p2_ref[1:2, :] * lax.rsqrt(var2 + 1e-5)
    sh2 = p2_ref[2:3, :] - mu2 * sc2

    def tile_bn2(t, _):
        o = pl.multiple_of(t * _T, _T)
        h2 = out_ref[pl.ds(o, _T), :]
        out_ref[pl.ds(o, _T), :] = jnp.maximum(h2 * sc2 + sh2, 0.0)
        return 0

    lax.fori_loop(0, _NT, tile_bn2, 0)


def kernel(x, pos, batch, x_skip, pos_skip, batch_skip,
           W1, b1, g1, be1, W2, b2, g2, be2):
    pos_t = pos.T
    batch_i = batch.astype(jnp.int32)
    batch_row = batch_i.reshape(1, _NC)
    batch_col = batch_skip.astype(jnp.int32).reshape(_NF, 1)
    x_hi = x.astype(jnp.bfloat16)
    x_lo = (x - x_hi.astype(jnp.float32)).astype(jnp.bfloat16)
    w1a, w1b = W1[:_DX], W1[_DX:]
    p1 = jnp.stack([b1, g1, be1])
    p2 = jnp.stack([b2, g2, be2])

    tids = jnp.arange(_NT)
    blo = batch_skip[tids * _T]
    bhi = batch_skip[tids * _T + (_T - 1)]
    clo = jnp.searchsorted(batch_i, blo.astype(jnp.int32), side="left")
    chi = jnp.searchsorted(batch_i, bhi.astype(jnp.int32), side="right")
    cbase = ((clo // _W) * _W).astype(jnp.int32)
    nch = ((chi.astype(jnp.int32) - cbase + _W - 1) // _W)

    h = pl.pallas_call(
        _body,
        out_shape=jax.ShapeDtypeStruct((_NF, _DOUT), jnp.float32),
        in_specs=[pl.BlockSpec(memory_space=pltpu.VMEM)] * 12
        + [pl.BlockSpec(memory_space=pltpu.SMEM)] * 2,
        out_specs=pl.BlockSpec(memory_space=pltpu.VMEM),
        scratch_shapes=[pltpu.VMEM((_NF, _DOUT), jnp.float32),
                        pltpu.VMEM((_T, _MAXCH * _W), jnp.float32)],
    )(pos_t, batch_row, x_hi, x_lo, pos_skip, batch_col, x_skip,
      w1a, w1b, p1, W2, p2, cbase, nch)
    return (h, pos_skip, batch_skip)
